# Initial kernel scaffold; baseline (speedup 1.0000x reference)
import jax
import jax.numpy as jnp
from jax.experimental import pallas as pl


def kernel(logits, labels):
    raise NotImplementedError("write your pallas kernel here")


# speedup vs baseline: 1.5529x; 1.5529x over previous
import functools
import math

import jax
import jax.numpy as jnp
from jax.experimental import pallas as pl
from jax.experimental.pallas import tpu as pltpu

_S = 64.0
_RB = 8


def _acos(x):
    p = jnp.float32(-0.0012624911)
    p = p * x + jnp.float32(0.0066700901)
    p = p * x - jnp.float32(0.0170881256)
    p = p * x + jnp.float32(0.0308918810)
    p = p * x - jnp.float32(0.0501743046)
    p = p * x + jnp.float32(0.0889789874)
    p = p * x - jnp.float32(0.2145988016)
    p = p * x + jnp.float32(1.5707963050)
    return jnp.sqrt(1.0 - x) * p


def _main_body(lab_ref, x_ref, out_ref, tgt_ref, win_ref, sum_ref):
    i = pl.program_id(0)
    x = x_ref[...]
    out_ref[...] = x * _S

    @pl.when(i == 0)
    def _():
        sum_ref[0, 0] = 0.0

    sum_ref[0, 0] += jnp.sum(x)

    tiles = []
    lanes = []
    for r in range(_RB):
        col = lab_ref[i * _RB + r]
        col_tile = pl.multiple_of((col // 128) * 128, 128)
        tiles.append(x_ref[pl.ds(r, 1), pl.ds(col_tile, 128)])
        lanes.append(col - col_tile)
    win = jnp.concatenate(tiles, axis=0)
    win_ref[...] = win * _S
    lane = jnp.concatenate([jnp.full((1, 1), l, jnp.int32) for l in lanes], axis=0)
    lane_iota = jax.lax.broadcasted_iota(jnp.int32, (_RB, 128), 1)
    picked = jnp.where(lane_iota == lane, win, 0.0)
    tgt_ref[0, :, :] = jnp.sum(picked, axis=1, keepdims=True)


def _fixup_body(out_in, tgt_ref, win_ref, lab2_ref, tot_ref, lab_ref, out_hbm,
                blend_ref, sem):
    del out_in
    b, n = out_hbm.shape
    t = tgt_ref[...]
    pos_sum = jnp.sum(t)
    total = tot_ref[0, 0]
    pos_mean = pos_sum / b
    neg_mean = (total - pos_sum) / (b * (n - 1))
    avg_p_theta = _acos(pos_mean)
    c = jnp.float32(math.log(n - 1) / _S)
    d_m = jnp.log(_acos(neg_mean + c) / math.pi) / jnp.log(avg_p_theta / math.pi)
    theta = _acos(t)
    ratio = theta * jnp.float32(1.0 / math.pi)
    warped = jnp.exp(d_m * jnp.log(ratio)) * jnp.float32(math.pi)
    final = jnp.cos(warped) * _S

    lane = jax.lax.rem(lab2_ref[...], jnp.int32(128))
    lane_iota = jax.lax.broadcasted_iota(jnp.int32, (b, 128), 1)
    blend_ref[...] = jnp.where(lane_iota == lane, final, win_ref[...])

    def _copy(i):
        col = lab_ref[i]
        col_tile = pl.multiple_of((col // 128) * 128, 128)
        return pltpu.make_async_copy(
            blend_ref.at[pl.ds(i, 1), :],
            out_hbm.at[pl.ds(i, 1), pl.ds(col_tile, 128)],
            sem,
        )

    def _start(i, _):
        _copy(i).start()
        return 0

    def _wait(i, _):
        _copy(i).wait()
        return 0

    jax.lax.fori_loop(0, b, _start, 0)
    jax.lax.fori_loop(0, b, _wait, 0)


@jax.jit
def kernel(logits, labels):
    b, n = logits.shape
    nb = b // _RB
    out0, tgt3, wins, total = pl.pallas_call(
        _main_body,
        grid=(nb,),
        in_specs=[
            pl.BlockSpec(memory_space=pltpu.SMEM),
            pl.BlockSpec((_RB, n), lambda i: (i, 0)),
        ],
        out_specs=[
            pl.BlockSpec((_RB, n), lambda i: (i, 0)),
            pl.BlockSpec((1, _RB, 1), lambda i: (i, 0, 0)),
            pl.BlockSpec((_RB, 128), lambda i: (i, 0)),
            pl.BlockSpec(memory_space=pltpu.SMEM),
        ],
        out_shape=[
            jax.ShapeDtypeStruct((b, n), jnp.float32),
            jax.ShapeDtypeStruct((nb, _RB, 1), jnp.float32),
            jax.ShapeDtypeStruct((b, 128), jnp.float32),
            jax.ShapeDtypeStruct((1, 1), jnp.float32),
        ],
    )(labels, logits)

    tgt = tgt3.reshape(b, 1)
    lab2 = labels.reshape(b, 1)

    out = pl.pallas_call(
        _fixup_body,
        in_specs=[
            pl.BlockSpec(memory_space=pl.ANY),
            pl.BlockSpec(memory_space=pltpu.VMEM),
            pl.BlockSpec(memory_space=pltpu.VMEM),
            pl.BlockSpec(memory_space=pltpu.VMEM),
            pl.BlockSpec(memory_space=pltpu.SMEM),
            pl.BlockSpec(memory_space=pltpu.SMEM),
        ],
        out_specs=pl.BlockSpec(memory_space=pl.ANY),
        out_shape=jax.ShapeDtypeStruct((b, n), jnp.float32),
        input_output_aliases={0: 0},
        scratch_shapes=[
            pltpu.VMEM((b, 128), jnp.float32),
            pltpu.SemaphoreType.DMA,
        ],
    )(out0, tgt, wins, lab2, total, labels)
    return out
